# Initial kernel scaffold; baseline (speedup 1.0000x reference)
import jax
import jax.numpy as jnp
from jax.experimental import pallas as pl


def kernel(x, embed):
    raise NotImplementedError("write your pallas kernel here")


# speedup vs baseline: 2.1856x; 2.1856x over previous
import functools

import jax
import jax.numpy as jnp
from jax import lax
from jax.experimental import pallas as pl
from jax.experimental.pallas import tpu as pltpu
from jax.experimental.pallas import tpu_sc as plsc

ROWS = 32 * 1024
CODES = 8192
DIM = 32
ROW_TILE = 512


def _dist_argmax_body(x_ref, et_ref, e2_ref, x2_ref, dist_ref, idx_ref):
    x = x_ref[...]
    et = et_ref[...]
    e2 = e2_ref[...]
    x2 = x2_ref[...]
    xy2 = lax.dot_general(
        x, et, (((1,), (0,)), ((), ())),
        preferred_element_type=jnp.float32,
    )
    d2 = x2 + e2 - xy2
    dist = -jnp.sqrt(jnp.maximum(d2, 0.0))
    dist_ref[...] = dist
    m = jnp.max(dist, axis=1, keepdims=True)
    iota = lax.broadcasted_iota(jnp.int32, dist.shape, 1)
    idx = jnp.min(jnp.where(dist >= m, iota, CODES), axis=1)
    idx_ref[0, 0, :] = idx


def _dist_argmax(xf, et, e2, x2, interpret=False):
    n_tiles = ROWS // ROW_TILE
    return pl.pallas_call(
        _dist_argmax_body,
        grid=(n_tiles,),
        in_specs=[
            pl.BlockSpec((ROW_TILE, DIM), lambda i: (i, 0)),
            pl.BlockSpec((DIM, CODES), lambda i: (0, 0)),
            pl.BlockSpec((1, CODES), lambda i: (0, 0)),
            pl.BlockSpec((ROW_TILE, 1), lambda i: (i, 0)),
        ],
        out_specs=[
            pl.BlockSpec((ROW_TILE, CODES), lambda i: (i, 0)),
            pl.BlockSpec((1, 1, ROW_TILE), lambda i: (i, 0, 0)),
        ],
        out_shape=[
            jax.ShapeDtypeStruct((ROWS, CODES), jnp.float32),
            jax.ShapeDtypeStruct((n_tiles, 1, ROW_TILE), jnp.int32),
        ],
        compiler_params=pltpu.CompilerParams(
            dimension_semantics=("parallel",),
        ),
        interpret=interpret,
    )(xf, et, e2, x2)


GATHER_W = 128


def _sc_gather(table_pad, idx_flat):
    info = plsc.get_sparse_core_info()
    nc, ns = info.num_cores, info.num_subcores
    nw = nc * ns
    rows_per_worker = ROWS // nw
    chunk = 128
    n_chunks = rows_per_worker // chunk
    idx3 = idx_flat.reshape(nw, n_chunks, chunk)

    @functools.partial(
        pl.kernel,
        mesh=plsc.VectorSubcoreMesh(core_axis_name="c", subcore_axis_name="s"),
        out_type=jax.ShapeDtypeStruct((ROWS, GATHER_W), jnp.float32),
        scratch_types=[
            pltpu.VMEM((n_chunks, chunk), jnp.int32),
            pltpu.VMEM((chunk, GATHER_W), jnp.float32),
            pltpu.SemaphoreType.DMA,
        ],
    )
    def gather_kernel(table_hbm, idx_hbm, out_hbm, idx_v, rows_v, sem):
        wid = lax.axis_index("s") * nc + lax.axis_index("c")
        pltpu.sync_copy(idx_hbm.at[wid], idx_v)
        base = wid * rows_per_worker
        for j in range(n_chunks):
            pltpu.async_copy(table_hbm.at[idx_v.at[j]], rows_v, sem).wait()
            pltpu.sync_copy(rows_v, out_hbm.at[pl.ds(base + j * chunk, chunk)])

    return gather_kernel(table_pad, idx3)


def kernel(x, embed):
    b, n, d = x.shape
    xf = x.reshape(ROWS, DIM)
    e = embed[0]
    et = (e + e).T
    e2 = jnp.sum(e * e, axis=-1)[None, :]
    x2 = jnp.sum(xf * xf, axis=-1)[:, None]
    dist_flat, idx_blk = _dist_argmax(xf, et, e2, x2)
    idx_flat = idx_blk.reshape(ROWS)
    e_pad = jnp.pad(e, ((0, 0), (0, GATHER_W - DIM)))
    quant_flat = _sc_gather(e_pad, idx_flat)
    quantize = quant_flat[:, :DIM].reshape(b, n, d)
    embed_ind = idx_flat.reshape(b, n)
    dist = dist_flat.reshape(b, n, CODES)
    return quantize, embed_ind, dist


# speedup vs baseline: 2.1913x; 1.0026x over previous
import functools

import jax
import jax.numpy as jnp
from jax import lax
from jax.experimental import pallas as pl
from jax.experimental.pallas import tpu as pltpu
from jax.experimental.pallas import tpu_sc as plsc

ROWS = 32 * 1024
CODES = 8192
DIM = 32
ROW_TILE = 512


def _dist_argmax_body(x_ref, et_ref, e2_ref, x2_ref, dist_ref, idx_ref):
    x = x_ref[...]
    et = et_ref[...]
    e2 = e2_ref[...]
    x2 = x2_ref[...]
    xy2 = lax.dot_general(
        x, et, (((1,), (0,)), ((), ())),
        preferred_element_type=jnp.float32,
    )
    d2 = x2 + e2 - xy2
    dist = -jnp.sqrt(jnp.maximum(d2, 0.0))
    dist_ref[...] = dist
    m = jnp.max(dist, axis=1, keepdims=True)
    iota = lax.broadcasted_iota(jnp.int32, dist.shape, 1)
    idx = jnp.min(jnp.where(dist >= m, iota, CODES), axis=1)
    idx_ref[0, 0, :] = idx


def _dist_argmax(xf, et, e2, x2, interpret=False):
    n_tiles = ROWS // ROW_TILE
    return pl.pallas_call(
        _dist_argmax_body,
        grid=(n_tiles,),
        in_specs=[
            pl.BlockSpec((ROW_TILE, DIM), lambda i: (i, 0)),
            pl.BlockSpec((DIM, CODES), lambda i: (0, 0)),
            pl.BlockSpec((1, CODES), lambda i: (0, 0)),
            pl.BlockSpec((ROW_TILE, 1), lambda i: (i, 0)),
        ],
        out_specs=[
            pl.BlockSpec((ROW_TILE, CODES), lambda i: (i, 0)),
            pl.BlockSpec((1, 1, ROW_TILE), lambda i: (i, 0, 0)),
        ],
        out_shape=[
            jax.ShapeDtypeStruct((ROWS, CODES), jnp.float32),
            jax.ShapeDtypeStruct((n_tiles, 1, ROW_TILE), jnp.int32),
        ],
        compiler_params=pltpu.CompilerParams(
            dimension_semantics=("parallel",),
        ),
        interpret=interpret,
    )(xf, et, e2, x2)


GATHER_W = 128


def _sc_gather(table_pad, idx_flat):
    info = plsc.get_sparse_core_info()
    nc, ns = info.num_cores, info.num_subcores
    nw = nc * ns
    rows_per_worker = ROWS // nw
    chunk = 128
    n_chunks = rows_per_worker // chunk
    idx3 = idx_flat.reshape(nw, n_chunks, chunk)

    @functools.partial(
        pl.kernel,
        mesh=plsc.VectorSubcoreMesh(core_axis_name="c", subcore_axis_name="s"),
        out_type=jax.ShapeDtypeStruct((ROWS, GATHER_W), jnp.float32),
        scratch_types=[
            pltpu.VMEM((n_chunks, chunk), jnp.int32),
            pltpu.VMEM((chunk, GATHER_W), jnp.float32),
            pltpu.VMEM((chunk, GATHER_W), jnp.float32),
            pltpu.SemaphoreType.DMA,
            pltpu.SemaphoreType.DMA,
        ],
    )
    def gather_kernel(table_hbm, idx_hbm, out_hbm, idx_v, rows_a, rows_b, sem_a, sem_b):
        wid = lax.axis_index("s") * nc + lax.axis_index("c")
        pltpu.sync_copy(idx_hbm.at[wid], idx_v)
        base = wid * rows_per_worker
        bufs = (rows_a, rows_b)
        sems = (sem_a, sem_b)
        pltpu.async_copy(table_hbm.at[idx_v.at[0]], bufs[0], sems[0])
        for j in range(n_chunks):
            if j + 1 < n_chunks:
                pltpu.async_copy(
                    table_hbm.at[idx_v.at[j + 1]], bufs[(j + 1) % 2], sems[(j + 1) % 2])
            pltpu.make_async_copy(
                table_hbm.at[idx_v.at[j]], bufs[j % 2], sems[j % 2]).wait()
            pltpu.sync_copy(bufs[j % 2],
                            out_hbm.at[pl.ds(base + j * chunk, chunk)])

    return gather_kernel(table_pad, idx3)


def kernel(x, embed):
    b, n, d = x.shape
    xf = x.reshape(ROWS, DIM)
    e = embed[0]
    et = (e + e).T
    e2 = jnp.sum(e * e, axis=-1)[None, :]
    x2 = jnp.sum(xf * xf, axis=-1)[:, None]
    dist_flat, idx_blk = _dist_argmax(xf, et, e2, x2)
    idx_flat = idx_blk.reshape(ROWS)
    e_pad = jnp.pad(e, ((0, 0), (0, GATHER_W - DIM)))
    quant_flat = _sc_gather(e_pad, idx_flat)
    quantize = quant_flat[:, :DIM].reshape(b, n, d)
    embed_ind = idx_flat.reshape(b, n)
    dist = dist_flat.reshape(b, n, CODES)
    return quantize, embed_ind, dist
